# Initial kernel scaffold; baseline (speedup 1.0000x reference)
import functools

import jax
import jax.numpy as jnp
from jax import lax
from jax.experimental import pallas as pl
from jax.experimental.pallas import tpu as pltpu
from jax.experimental.pallas import tpu_sc as plsc

N = 10000
E = 320000
C_IN = 128
C_HID = 128
C_OUT = 64

NC = 2
NS = 16
NW = NC * NS
EPW = E // NW
CH = 80
NCH = EPW // CH
GRP = 5
NGRP = NCH // GRP
RPT = N // NS

_f32 = jnp.float32
_zero16 = jnp.zeros((16,), _f32)
_one16 = jnp.ones((16,), _f32)


def _mesh():
    return plsc.VectorSubcoreMesh(core_axis_name="c", subcore_axis_name="s")


def _fill_zero(ref, rows, cols):
    def body(i, _):
        for j in range(cols // 16):
            ref[i, pl.ds(j * 16, 16)] = _zero16
        return 0
    lax.fori_loop(0, rows, body, 0)


DEGW = 16


@functools.partial(
    pl.kernel,
    out_type=jax.ShapeDtypeStruct((NC, N, DEGW), _f32),
    mesh=_mesh(),
    scratch_types=[
        pltpu.VMEM((NCH, CH), jnp.int32),
        pltpu.VMEM((CH, DEGW), _f32),
        pltpu.VMEM((RPT, DEGW), _f32),
        pltpu.VMEM_SHARED((N, DEGW), _f32),
    ],
)
def _deg_kernel(col_hbm, out_hbm, col_v, ones_v, zbuf_v, acc_sh):
    cid = lax.axis_index("c")
    sid = lax.axis_index("s")
    wid = cid * NS + sid

    pltpu.sync_copy(col_hbm.at[pl.ds(wid * NCH, NCH)], col_v)

    def fill_ones(i, _):
        ones_v[i, :] = _one16
        return 0
    lax.fori_loop(0, CH, fill_ones, 0)

    _fill_zero(zbuf_v, RPT, DEGW)
    pltpu.sync_copy(zbuf_v, acc_sh.at[pl.ds(sid * RPT, RPT)])
    plsc.subcore_barrier()

    def body(j, _):
        pltpu.sync_copy(ones_v, acc_sh.at[col_v.at[j]], add=True)
        return 0
    lax.fori_loop(0, NCH, body, 0)

    plsc.subcore_barrier()
    pltpu.sync_copy(acc_sh.at[pl.ds(sid * RPT, RPT)],
                    out_hbm.at[cid, pl.ds(sid * RPT, RPT)])


def _make_scatter(C):
    @functools.partial(
        pl.kernel,
        out_type=jax.ShapeDtypeStruct((NC, N, C), _f32),
        mesh=_mesh(),
        scratch_types=[
            pltpu.VMEM((NCH, CH), jnp.int32),
            pltpu.VMEM((NCH, CH), jnp.int32),
            pltpu.VMEM((GRP, CH, C), _f32),
            pltpu.VMEM((RPT, C), _f32),
            pltpu.VMEM_SHARED((N, C), _f32),
            pltpu.SemaphoreType.DMA,
        ],
    )
    def _scatter_kernel(row_hbm, col_hbm, tab_hbm, out_hbm,
                        row_v, col_v, buf_v, zbuf_v, acc_sh, sem):
        cid = lax.axis_index("c")
        sid = lax.axis_index("s")
        wid = cid * NS + sid

        pltpu.sync_copy(row_hbm.at[pl.ds(wid * NCH, NCH)], row_v)
        pltpu.sync_copy(col_hbm.at[pl.ds(wid * NCH, NCH)], col_v)

        _fill_zero(zbuf_v, RPT, C)
        pltpu.sync_copy(zbuf_v, acc_sh.at[pl.ds(sid * RPT, RPT)])
        plsc.subcore_barrier()

        def group(g, _):
            base = g * GRP
            for b in range(GRP):
                pltpu.async_copy(tab_hbm.at[row_v.at[base + b]],
                                 buf_v.at[b], sem)
            for b in range(GRP):
                pltpu.make_async_copy(tab_hbm.at[row_v.at[base + b]],
                                      buf_v.at[b], sem).wait()
                pltpu.sync_copy(buf_v.at[b], acc_sh.at[col_v.at[base + b]],
                                add=True)
            return 0
        lax.fori_loop(0, NGRP, group, 0)

        plsc.subcore_barrier()
        pltpu.sync_copy(acc_sh.at[pl.ds(sid * RPT, RPT)],
                        out_hbm.at[cid, pl.ds(sid * RPT, RPT)])

    return _scatter_kernel


_scatter_hid = _make_scatter(C_HID)
_scatter_out = _make_scatter(C_OUT)


RB = 400
NRB = N // RB


def _tc_a_body(x_ref, w1_ref, p0_ref, p1_ref, hs_ref, hh_ref, dis_ref):
    deg = 1.0 + p0_ref[:, 0:1] + p1_ref[:, 0:1]
    dis = lax.rsqrt(deg)
    h = jnp.dot(x_ref[...], w1_ref[...], preferred_element_type=_f32)
    hs_ref[...] = h * dis
    hh_ref[...] = h * (dis * dis)
    dis_ref[...] = jnp.broadcast_to(dis, (RB, 8))


@jax.jit
def _tc_a(x, W1, p0, p1):
    return pl.pallas_call(
        _tc_a_body,
        grid=(NRB,),
        in_specs=[
            pl.BlockSpec((RB, C_IN), lambda i: (i, 0)),
            pl.BlockSpec((C_IN, C_HID), lambda i: (0, 0)),
            pl.BlockSpec((RB, DEGW), lambda i: (i, 0)),
            pl.BlockSpec((RB, DEGW), lambda i: (i, 0)),
        ],
        out_specs=[
            pl.BlockSpec((RB, C_HID), lambda i: (i, 0)),
            pl.BlockSpec((RB, C_HID), lambda i: (i, 0)),
            pl.BlockSpec((RB, 8), lambda i: (i, 0)),
        ],
        out_shape=[
            jax.ShapeDtypeStruct((N, C_HID), _f32),
            jax.ShapeDtypeStruct((N, C_HID), _f32),
            jax.ShapeDtypeStruct((N, 8), _f32),
        ],
    )(x, W1, p0, p1)


def _tc_b_body(a0_ref, a1_ref, hh_ref, dis_ref, b1_ref, w2_ref,
               hs2_ref, hh2_ref):
    dis = dis_ref[:, 0:1]
    h1 = dis * (a0_ref[...] + a1_ref[...]) + hh_ref[...] + b1_ref[...]
    h1 = jnp.maximum(h1, 0.0)
    h2 = jnp.dot(h1, w2_ref[...], preferred_element_type=_f32)
    hs2_ref[...] = h2 * dis
    hh2_ref[...] = h2 * (dis * dis)


@jax.jit
def _tc_b(a0, a1, hh, dis, b1, W2):
    return pl.pallas_call(
        _tc_b_body,
        grid=(NRB,),
        in_specs=[
            pl.BlockSpec((RB, C_HID), lambda i: (i, 0)),
            pl.BlockSpec((RB, C_HID), lambda i: (i, 0)),
            pl.BlockSpec((RB, C_HID), lambda i: (i, 0)),
            pl.BlockSpec((RB, 8), lambda i: (i, 0)),
            pl.BlockSpec((1, C_HID), lambda i: (0, 0)),
            pl.BlockSpec((C_HID, C_OUT), lambda i: (0, 0)),
        ],
        out_specs=[
            pl.BlockSpec((RB, C_OUT), lambda i: (i, 0)),
            pl.BlockSpec((RB, C_OUT), lambda i: (i, 0)),
        ],
        out_shape=[
            jax.ShapeDtypeStruct((N, C_OUT), _f32),
            jax.ShapeDtypeStruct((N, C_OUT), _f32),
        ],
    )(a0, a1, hh, dis, b1, W2)


def _tc_c_body(a0_ref, a1_ref, hh2_ref, dis_ref, b2_ref, out_ref):
    dis = dis_ref[:, 0:1]
    out_ref[...] = (dis * (a0_ref[...] + a1_ref[...]) + hh2_ref[...]
                    + b2_ref[...])


@jax.jit
def _tc_c(a0, a1, hh2, dis, b2):
    return pl.pallas_call(
        _tc_c_body,
        grid=(NRB,),
        in_specs=[
            pl.BlockSpec((RB, C_OUT), lambda i: (i, 0)),
            pl.BlockSpec((RB, C_OUT), lambda i: (i, 0)),
            pl.BlockSpec((RB, C_OUT), lambda i: (i, 0)),
            pl.BlockSpec((RB, 8), lambda i: (i, 0)),
            pl.BlockSpec((1, C_OUT), lambda i: (0, 0)),
        ],
        out_specs=pl.BlockSpec((RB, C_OUT), lambda i: (i, 0)),
        out_shape=jax.ShapeDtypeStruct((N, C_OUT), _f32),
    )(a0, a1, hh2, dis, b2)


@jax.jit
def kernel(x, edge_index, W1, b1, W2, b2):
    row = edge_index[0].astype(jnp.int32).reshape(E // CH, CH)
    col = edge_index[1].astype(jnp.int32).reshape(E // CH, CH)

    degp = _deg_kernel(col)
    hs, hh, dis = _tc_a(x, W1, degp[0], degp[1])
    acc = _scatter_hid(row, col, hs)
    hs2, hh2 = _tc_b(acc[0], acc[1], hh, dis, b1.reshape(1, C_HID), W2)
    acc2 = _scatter_out(row, col, hs2)
    return _tc_c(acc2[0], acc2[1], hh2, dis, b2.reshape(1, C_OUT))


# speedup vs baseline: 20.4992x; 20.4992x over previous
import functools

import jax
import jax.numpy as jnp
from jax import lax
from jax.experimental import pallas as pl
from jax.experimental.pallas import tpu as pltpu
from jax.experimental.pallas import tpu_sc as plsc

N = 10000
E = 320000
C_IN = 128
C_HID = 128
C_OUT = 64
FS = 32

NC = 2
NS = 16
NW = NC * NS

CH = 80
GRP = 5

EPT = E // NS
NCH = EPT // CH
NGRP = NCH // GRP

EPW = E // NW
NCHD = EPW // CH

BR = 624
XR = N - NS * BR
ZR = 208

DEGW = 16

_f32 = jnp.float32


def _mesh():
    return plsc.VectorSubcoreMesh(core_axis_name="c", subcore_axis_name="s")


_sc_params = pltpu.CompilerParams(use_tc_tiling_on_sc=False)


def _fill_zero(ref, rows, cols):
    zero16 = jnp.zeros((16,), _f32)

    def body(i, _):
        for j in range(cols // 16):
            ref[i, pl.ds(j * 16, 16)] = zero16
        return 0
    lax.fori_loop(0, rows, body, 0)


def _zero_and_barrier(zbuf_v, acc_sh, sid):
    for k in range(BR // ZR):
        pltpu.sync_copy(zbuf_v, acc_sh.at[pl.ds(sid * BR + k * ZR, ZR)])

    @pl.when(sid == NS - 1)
    def _():
        pltpu.sync_copy(zbuf_v.at[pl.ds(0, XR)],
                        acc_sh.at[pl.ds(NS * BR, XR)])
    plsc.subcore_barrier()


def _copy_out(acc_sh, out_slice, sid):
    pltpu.sync_copy(acc_sh.at[pl.ds(sid * BR, BR)],
                    out_slice.at[pl.ds(sid * BR, BR)])

    @pl.when(sid == NS - 1)
    def _():
        pltpu.sync_copy(acc_sh.at[pl.ds(NS * BR, XR)],
                        out_slice.at[pl.ds(NS * BR, XR)])


@functools.partial(
    pl.kernel,
    out_type=jax.ShapeDtypeStruct((NC, N, DEGW), _f32),
    mesh=_mesh(),
    compiler_params=_sc_params,
    scratch_types=[
        pltpu.VMEM((NCHD, CH), jnp.int32),
        pltpu.VMEM((CH, DEGW), _f32),
        pltpu.VMEM((ZR, DEGW), _f32),
        pltpu.VMEM_SHARED((N, DEGW), _f32),
    ],
)
def _deg_kernel(col_hbm, out_hbm, col_v, ones_v, zbuf_v, acc_sh):
    cid = lax.axis_index("c")
    sid = lax.axis_index("s")
    wid = cid * NS + sid

    pltpu.sync_copy(col_hbm.at[wid], col_v)

    one16 = jnp.ones((16,), _f32)

    def fill_ones(i, _):
        ones_v[i, :] = one16
        return 0
    lax.fori_loop(0, CH, fill_ones, 0)

    _fill_zero(zbuf_v, ZR, DEGW)
    _zero_and_barrier(zbuf_v, acc_sh, sid)

    def body(j, _):
        pltpu.sync_copy(ones_v, acc_sh.at[col_v.at[j]], add=True)
        return 0
    lax.fori_loop(0, NCHD, body, 0)

    plsc.subcore_barrier()
    _copy_out(acc_sh, out_hbm.at[cid], sid)


def _make_scatter(NBLK):
    @functools.partial(
        pl.kernel,
        out_type=jax.ShapeDtypeStruct((NBLK, N, FS), _f32),
        mesh=_mesh(),
        compiler_params=_sc_params,
        scratch_types=[
            pltpu.VMEM((NCH, CH), jnp.int32),
            pltpu.VMEM((NCH, CH), jnp.int32),
            pltpu.VMEM((GRP, CH, FS), _f32),
            pltpu.VMEM((ZR, FS), _f32),
            pltpu.VMEM_SHARED((N, FS), _f32),
            pltpu.SemaphoreType.DMA,
        ],
    )
    def _scatter_kernel(row_hbm, col_hbm, tab_hbm, out_hbm,
                        row_v, col_v, buf_v, zbuf_v, acc_sh, sem):
        cid = lax.axis_index("c")
        sid = lax.axis_index("s")

        pltpu.sync_copy(row_hbm.at[sid], row_v)
        pltpu.sync_copy(col_hbm.at[sid], col_v)
        _fill_zero(zbuf_v, ZR, FS)

        for p in range(NBLK // NC):
            blk = NC * p + cid
            tab = tab_hbm.at[blk]
            _zero_and_barrier(zbuf_v, acc_sh, sid)

            def group(g, _):
                base = g * GRP
                for b in range(GRP):
                    pltpu.async_copy(tab.at[row_v.at[base + b]],
                                     buf_v.at[b], sem)
                for b in range(GRP):
                    pltpu.make_async_copy(tab.at[row_v.at[base + b]],
                                          buf_v.at[b], sem).wait()
                    pltpu.sync_copy(buf_v.at[b],
                                    acc_sh.at[col_v.at[base + b]],
                                    add=True)
                return 0
            lax.fori_loop(0, NGRP, group, 0)

            plsc.subcore_barrier()
            _copy_out(acc_sh, out_hbm.at[blk], sid)
            plsc.subcore_barrier()

    return _scatter_kernel


_scatter_hid = _make_scatter(C_HID // FS)
_scatter_out = _make_scatter(C_OUT // FS)


RB = 400
NRB = N // RB


def _tc_a_body(x_ref, w1_ref, p0_ref, p1_ref,
               t0_ref, t1_ref, t2_ref, t3_ref, hh_ref, dis_ref):
    deg = 1.0 + p0_ref[:, 0:1] + p1_ref[:, 0:1]
    dis = lax.rsqrt(deg)
    h = jnp.dot(x_ref[...], w1_ref[...], preferred_element_type=_f32)
    hs = h * dis
    t0_ref[...] = hs[:, 0 * FS:1 * FS]
    t1_ref[...] = hs[:, 1 * FS:2 * FS]
    t2_ref[...] = hs[:, 2 * FS:3 * FS]
    t3_ref[...] = hs[:, 3 * FS:4 * FS]
    hh_ref[...] = h * (dis * dis)
    dis_ref[...] = jnp.broadcast_to(dis, (RB, 8))


@jax.jit
def _tc_a(x, W1, p0, p1):
    fs_spec = pl.BlockSpec((RB, FS), lambda i: (i, 0))
    return pl.pallas_call(
        _tc_a_body,
        grid=(NRB,),
        in_specs=[
            pl.BlockSpec((RB, C_IN), lambda i: (i, 0)),
            pl.BlockSpec((C_IN, C_HID), lambda i: (0, 0)),
            pl.BlockSpec((RB, DEGW), lambda i: (i, 0)),
            pl.BlockSpec((RB, DEGW), lambda i: (i, 0)),
        ],
        out_specs=[fs_spec, fs_spec, fs_spec, fs_spec,
                   pl.BlockSpec((RB, C_HID), lambda i: (i, 0)),
                   pl.BlockSpec((RB, 8), lambda i: (i, 0))],
        out_shape=[jax.ShapeDtypeStruct((N, FS), _f32)] * 4 +
                  [jax.ShapeDtypeStruct((N, C_HID), _f32),
                   jax.ShapeDtypeStruct((N, 8), _f32)],
    )(x, W1, p0, p1)


def _tc_b_body(a0_ref, a1_ref, a2_ref, a3_ref, hh_ref, dis_ref, b1_ref,
               w2_ref, t0_ref, t1_ref, hh2_ref):
    dis = dis_ref[:, 0:1]
    agg = jnp.concatenate(
        [a0_ref[...], a1_ref[...], a2_ref[...], a3_ref[...]], axis=1)
    h1 = dis * agg + hh_ref[...] + b1_ref[...]
    h1 = jnp.maximum(h1, 0.0)
    h2 = jnp.dot(h1, w2_ref[...], preferred_element_type=_f32)
    hs2 = h2 * dis
    t0_ref[...] = hs2[:, 0 * FS:1 * FS]
    t1_ref[...] = hs2[:, 1 * FS:2 * FS]
    hh2_ref[...] = h2 * (dis * dis)


@jax.jit
def _tc_b(a0, a1, a2, a3, hh, dis, b1, W2):
    fs_spec = pl.BlockSpec((RB, FS), lambda i: (i, 0))
    return pl.pallas_call(
        _tc_b_body,
        grid=(NRB,),
        in_specs=[fs_spec, fs_spec, fs_spec, fs_spec,
                  pl.BlockSpec((RB, C_HID), lambda i: (i, 0)),
                  pl.BlockSpec((RB, 8), lambda i: (i, 0)),
                  pl.BlockSpec((1, C_HID), lambda i: (0, 0)),
                  pl.BlockSpec((C_HID, C_OUT), lambda i: (0, 0))],
        out_specs=[fs_spec, fs_spec,
                   pl.BlockSpec((RB, C_OUT), lambda i: (i, 0))],
        out_shape=[jax.ShapeDtypeStruct((N, FS), _f32)] * 2 +
                  [jax.ShapeDtypeStruct((N, C_OUT), _f32)],
    )(a0, a1, a2, a3, hh, dis, b1, W2)


def _tc_c_body(a0_ref, a1_ref, hh2_ref, dis_ref, b2_ref, out_ref):
    dis = dis_ref[:, 0:1]
    agg = jnp.concatenate([a0_ref[...], a1_ref[...]], axis=1)
    out_ref[...] = dis * agg + hh2_ref[...] + b2_ref[...]


@jax.jit
def _tc_c(a0, a1, hh2, dis, b2):
    fs_spec = pl.BlockSpec((RB, FS), lambda i: (i, 0))
    return pl.pallas_call(
        _tc_c_body,
        grid=(NRB,),
        in_specs=[fs_spec, fs_spec,
                  pl.BlockSpec((RB, C_OUT), lambda i: (i, 0)),
                  pl.BlockSpec((RB, 8), lambda i: (i, 0)),
                  pl.BlockSpec((1, C_OUT), lambda i: (0, 0))],
        out_specs=pl.BlockSpec((RB, C_OUT), lambda i: (i, 0)),
        out_shape=jax.ShapeDtypeStruct((N, C_OUT), _f32),
    )(a0, a1, hh2, dis, b2)


@jax.jit
def kernel(x, edge_index, W1, b1, W2, b2):
    row = edge_index[0].astype(jnp.int32)
    col = edge_index[1].astype(jnp.int32)
    row16 = row.reshape(NS, NCH, CH)
    col16 = col.reshape(NS, NCH, CH)
    col32 = col.reshape(NW, NCHD, CH)

    degp = _deg_kernel(col32)
    t0, t1, t2, t3, hh, dis = _tc_a(x, W1, degp[0], degp[1])
    acc = _scatter_hid(row16, col16, jnp.stack([t0, t1, t2, t3]))
    s0, s1, hh2 = _tc_b(acc[0], acc[1], acc[2], acc[3], hh, dis,
                        b1.reshape(1, C_HID), W2)
    acc2 = _scatter_out(row16, col16, jnp.stack([s0, s1]))
    return _tc_c(acc2[0], acc2[1], hh2, dis, b2.reshape(1, C_OUT))


# speedup vs baseline: 23.0223x; 1.1231x over previous
import functools

import jax
import jax.numpy as jnp
from jax import lax
from jax.experimental import pallas as pl
from jax.experimental.pallas import tpu as pltpu
from jax.experimental.pallas import tpu_sc as plsc

N = 10000
E = 320000
C_IN = 128
C_HID = 128
C_OUT = 64
FS = 32

NC = 2
NS = 16
NW = NC * NS

CH = 80
GRP = 5

EPT = E // NS
NCH = EPT // CH
NGRP = NCH // GRP

EPW = E // NW
NCHD = EPW // CH

BR = 624
XR = N - NS * BR
ZR = 208

DEGW = 16

_f32 = jnp.float32


def _mesh():
    return plsc.VectorSubcoreMesh(core_axis_name="c", subcore_axis_name="s")


_sc_params = pltpu.CompilerParams(use_tc_tiling_on_sc=False)


def _fill_zero(ref, rows, cols):
    zero16 = jnp.zeros((16,), _f32)

    def body(i, _):
        for j in range(cols // 16):
            ref[i, pl.ds(j * 16, 16)] = zero16
        return 0
    lax.fori_loop(0, rows, body, 0)


def _zero_and_barrier(zbuf_v, acc_sh, sid):
    for k in range(BR // ZR):
        pltpu.sync_copy(zbuf_v, acc_sh.at[pl.ds(sid * BR + k * ZR, ZR)])

    @pl.when(sid == NS - 1)
    def _():
        pltpu.sync_copy(zbuf_v.at[pl.ds(0, XR)],
                        acc_sh.at[pl.ds(NS * BR, XR)])
    plsc.subcore_barrier()


def _copy_out(acc_sh, out_slice, sid):
    pltpu.sync_copy(acc_sh.at[pl.ds(sid * BR, BR)],
                    out_slice.at[pl.ds(sid * BR, BR)])

    @pl.when(sid == NS - 1)
    def _():
        pltpu.sync_copy(acc_sh.at[pl.ds(NS * BR, XR)],
                        out_slice.at[pl.ds(NS * BR, XR)])


@functools.partial(
    pl.kernel,
    out_type=jax.ShapeDtypeStruct((NC, N, DEGW), _f32),
    mesh=_mesh(),
    compiler_params=_sc_params,
    scratch_types=[
        pltpu.VMEM((NCHD, CH), jnp.int32),
        pltpu.VMEM((CH, DEGW), _f32),
        pltpu.VMEM((ZR, DEGW), _f32),
        pltpu.VMEM_SHARED((N, DEGW), _f32),
    ],
)
def _deg_kernel(col_hbm, out_hbm, col_v, ones_v, zbuf_v, acc_sh):
    cid = lax.axis_index("c")
    sid = lax.axis_index("s")
    wid = cid * NS + sid

    pltpu.sync_copy(col_hbm.at[wid], col_v)

    one16 = jnp.ones((16,), _f32)

    def fill_ones(i, _):
        ones_v[i, :] = one16
        return 0
    lax.fori_loop(0, CH, fill_ones, 0)

    _fill_zero(zbuf_v, ZR, DEGW)
    _zero_and_barrier(zbuf_v, acc_sh, sid)

    def body(j, _):
        pltpu.sync_copy(ones_v, acc_sh.at[col_v.at[j]], add=True)
        return 0
    lax.fori_loop(0, NCHD, body, 0)

    plsc.subcore_barrier()
    _copy_out(acc_sh, out_hbm.at[cid], sid)


def _make_scatter(NBLK):
    @functools.partial(
        pl.kernel,
        out_type=jax.ShapeDtypeStruct((NBLK, N, FS), _f32),
        mesh=_mesh(),
        compiler_params=_sc_params,
        scratch_types=[
            pltpu.VMEM((NCH, CH), jnp.int32),
            pltpu.VMEM((NCH, CH), jnp.int32),
            pltpu.VMEM((2, GRP, CH, FS), _f32),
            pltpu.VMEM((ZR, FS), _f32),
            pltpu.VMEM_SHARED((N, FS), _f32),
            pltpu.SemaphoreType.DMA,
            pltpu.SemaphoreType.DMA,
        ],
    )
    def _scatter_kernel(row_hbm, col_hbm, tab_hbm, out_hbm,
                        row_v, col_v, buf_v, zbuf_v, acc_sh, semg, sems):
        cid = lax.axis_index("c")
        sid = lax.axis_index("s")

        pltpu.sync_copy(row_hbm.at[sid], row_v)
        pltpu.sync_copy(col_hbm.at[sid], col_v)
        _fill_zero(zbuf_v, ZR, FS)

        for p in range(NBLK // NC):
            blk = NC * p + cid
            tab = tab_hbm.at[blk]
            _zero_and_barrier(zbuf_v, acc_sh, sid)

            def wait_scatters(g):
                half = lax.rem(g, 2)
                for b in range(GRP):
                    pltpu.make_async_copy(
                        buf_v.at[half, b],
                        acc_sh.at[col_v.at[g * GRP + b]], sems).wait()

            def group(g, _):
                half = lax.rem(g, 2)

                @pl.when(g >= 2)
                def _():
                    wait_scatters(g - 2)
                for b in range(GRP):
                    pltpu.async_copy(tab.at[row_v.at[g * GRP + b]],
                                     buf_v.at[half, b], semg)
                for b in range(GRP):
                    pltpu.make_async_copy(tab.at[row_v.at[g * GRP + b]],
                                          buf_v.at[half, b], semg).wait()
                    pltpu.async_copy(buf_v.at[half, b],
                                     acc_sh.at[col_v.at[g * GRP + b]],
                                     sems, add=True)
                return 0
            lax.fori_loop(0, NGRP, group, 0)
            wait_scatters(NGRP - 2)
            wait_scatters(NGRP - 1)

            plsc.subcore_barrier()
            _copy_out(acc_sh, out_hbm.at[blk], sid)
            plsc.subcore_barrier()

    return _scatter_kernel


_scatter_hid = _make_scatter(C_HID // FS)
_scatter_out = _make_scatter(C_OUT // FS)


RB = 400
NRB = N // RB


def _tc_a_body(x_ref, w1_ref, p0_ref, p1_ref,
               t0_ref, t1_ref, t2_ref, t3_ref, hh_ref, dis_ref):
    deg = 1.0 + p0_ref[:, 0:1] + p1_ref[:, 0:1]
    dis = lax.rsqrt(deg)
    h = jnp.dot(x_ref[...], w1_ref[...], preferred_element_type=_f32)
    hs = h * dis
    t0_ref[...] = hs[:, 0 * FS:1 * FS]
    t1_ref[...] = hs[:, 1 * FS:2 * FS]
    t2_ref[...] = hs[:, 2 * FS:3 * FS]
    t3_ref[...] = hs[:, 3 * FS:4 * FS]
    hh_ref[...] = h * (dis * dis)
    dis_ref[...] = jnp.broadcast_to(dis, (RB, 8))


@jax.jit
def _tc_a(x, W1, p0, p1):
    fs_spec = pl.BlockSpec((RB, FS), lambda i: (i, 0))
    return pl.pallas_call(
        _tc_a_body,
        grid=(NRB,),
        in_specs=[
            pl.BlockSpec((RB, C_IN), lambda i: (i, 0)),
            pl.BlockSpec((C_IN, C_HID), lambda i: (0, 0)),
            pl.BlockSpec((RB, DEGW), lambda i: (i, 0)),
            pl.BlockSpec((RB, DEGW), lambda i: (i, 0)),
        ],
        out_specs=[fs_spec, fs_spec, fs_spec, fs_spec,
                   pl.BlockSpec((RB, C_HID), lambda i: (i, 0)),
                   pl.BlockSpec((RB, 8), lambda i: (i, 0))],
        out_shape=[jax.ShapeDtypeStruct((N, FS), _f32)] * 4 +
                  [jax.ShapeDtypeStruct((N, C_HID), _f32),
                   jax.ShapeDtypeStruct((N, 8), _f32)],
    )(x, W1, p0, p1)


def _tc_b_body(a0_ref, a1_ref, a2_ref, a3_ref, hh_ref, dis_ref, b1_ref,
               w2_ref, t0_ref, t1_ref, hh2_ref):
    dis = dis_ref[:, 0:1]
    agg = jnp.concatenate(
        [a0_ref[...], a1_ref[...], a2_ref[...], a3_ref[...]], axis=1)
    h1 = dis * agg + hh_ref[...] + b1_ref[...]
    h1 = jnp.maximum(h1, 0.0)
    h2 = jnp.dot(h1, w2_ref[...], preferred_element_type=_f32)
    hs2 = h2 * dis
    t0_ref[...] = hs2[:, 0 * FS:1 * FS]
    t1_ref[...] = hs2[:, 1 * FS:2 * FS]
    hh2_ref[...] = h2 * (dis * dis)


@jax.jit
def _tc_b(a0, a1, a2, a3, hh, dis, b1, W2):
    fs_spec = pl.BlockSpec((RB, FS), lambda i: (i, 0))
    return pl.pallas_call(
        _tc_b_body,
        grid=(NRB,),
        in_specs=[fs_spec, fs_spec, fs_spec, fs_spec,
                  pl.BlockSpec((RB, C_HID), lambda i: (i, 0)),
                  pl.BlockSpec((RB, 8), lambda i: (i, 0)),
                  pl.BlockSpec((1, C_HID), lambda i: (0, 0)),
                  pl.BlockSpec((C_HID, C_OUT), lambda i: (0, 0))],
        out_specs=[fs_spec, fs_spec,
                   pl.BlockSpec((RB, C_OUT), lambda i: (i, 0))],
        out_shape=[jax.ShapeDtypeStruct((N, FS), _f32)] * 2 +
                  [jax.ShapeDtypeStruct((N, C_OUT), _f32)],
    )(a0, a1, a2, a3, hh, dis, b1, W2)


def _tc_c_body(a0_ref, a1_ref, hh2_ref, dis_ref, b2_ref, out_ref):
    dis = dis_ref[:, 0:1]
    agg = jnp.concatenate([a0_ref[...], a1_ref[...]], axis=1)
    out_ref[...] = dis * agg + hh2_ref[...] + b2_ref[...]


@jax.jit
def _tc_c(a0, a1, hh2, dis, b2):
    fs_spec = pl.BlockSpec((RB, FS), lambda i: (i, 0))
    return pl.pallas_call(
        _tc_c_body,
        grid=(NRB,),
        in_specs=[fs_spec, fs_spec,
                  pl.BlockSpec((RB, C_OUT), lambda i: (i, 0)),
                  pl.BlockSpec((RB, 8), lambda i: (i, 0)),
                  pl.BlockSpec((1, C_OUT), lambda i: (0, 0))],
        out_specs=pl.BlockSpec((RB, C_OUT), lambda i: (i, 0)),
        out_shape=jax.ShapeDtypeStruct((N, C_OUT), _f32),
    )(a0, a1, hh2, dis, b2)


@jax.jit
def kernel(x, edge_index, W1, b1, W2, b2):
    row = edge_index[0].astype(jnp.int32)
    col = edge_index[1].astype(jnp.int32)
    row16 = row.reshape(NS, NCH, CH)
    col16 = col.reshape(NS, NCH, CH)
    col32 = col.reshape(NW, NCHD, CH)

    degp = _deg_kernel(col32)
    t0, t1, t2, t3, hh, dis = _tc_a(x, W1, degp[0], degp[1])
    acc = _scatter_hid(row16, col16, jnp.stack([t0, t1, t2, t3]))
    s0, s1, hh2 = _tc_b(acc[0], acc[1], acc[2], acc[3], hh, dis,
                        b1.reshape(1, C_HID), W2)
    acc2 = _scatter_out(row16, col16, jnp.stack([s0, s1]))
    return _tc_c(acc2[0], acc2[1], hh2, dis, b2.reshape(1, C_OUT))
